# baseline (speedup 1.0000x reference)
import functools

import jax
import jax.numpy as jnp
from jax.experimental import pallas as pl
from jax.experimental.pallas import tpu as pltpu

NH = 512
DT = 1.0
N_NODES = 100000
N_ROWS = NH + 2

CHUNK = 64
_STARTS = list(range(0, N_ROWS, CHUNK))
_SIZES = [min(CHUNK, N_ROWS - s) for s in _STARTS]
N_CHUNKS = len(_STARTS)


def _step_kernel(t_ref, buf_ref, w_ref, out_ref, nx_ref,
                 x_s, r0_s, r1_s, copy_sems, row_sems, wr_sem):
    tt = t_ref[0, 0]
    bulk = []
    for i, (s, z) in enumerate(zip(_STARTS, _SIZES)):
        c = pltpu.make_async_copy(
            buf_ref.at[pl.ds(s, z)], out_ref.at[pl.ds(s, z)],
            copy_sems.at[i])
        c.start()
        bulk.append(c)
    cx = pltpu.make_async_copy(buf_ref.at[pl.ds(NH + tt, 1)], x_s,
                               row_sems.at[0])
    c0 = pltpu.make_async_copy(buf_ref.at[pl.ds(tt, 1)], r0_s,
                               row_sems.at[1])
    c1 = pltpu.make_async_copy(buf_ref.at[pl.ds(tt + 1, 1)], r1_s,
                               row_sems.at[2])
    cx.start()
    c0.start()
    c1.start()
    cx.wait()
    c0.wait()
    c1.wait()
    x = x_s[...]
    w = w_ref[...]
    d1 = 0.1 * (r0_s[...] - x)
    xi = jnp.tanh(x + DT * d1 + w)
    d2 = 0.1 * (r1_s[...] - xi)
    nx = jnp.tanh(x + DT * 0.5 * (d1 + d2) + w)
    nx_ref[...] = nx
    for c in bulk:
        c.wait()
    wr = pltpu.make_async_copy(nx_ref, out_ref.at[pl.ds(NH + tt + 1, 1)],
                               wr_sem)
    wr.start()
    wr.wait()


@functools.partial(jax.jit, static_argnames=())
def kernel(buf, dWt, t):
    w2d = dWt.reshape(1, N_NODES)
    out_buf, nx2d = pl.pallas_call(
        _step_kernel,
        in_specs=[
            pl.BlockSpec(memory_space=pltpu.SMEM),
            pl.BlockSpec(memory_space=pl.ANY),
            pl.BlockSpec(memory_space=pltpu.VMEM),
        ],
        out_specs=[
            pl.BlockSpec(memory_space=pl.ANY),
            pl.BlockSpec(memory_space=pltpu.VMEM),
        ],
        out_shape=[
            jax.ShapeDtypeStruct((N_ROWS, N_NODES), jnp.float32),
            jax.ShapeDtypeStruct((1, N_NODES), jnp.float32),
        ],
        scratch_shapes=[
            pltpu.VMEM((1, N_NODES), jnp.float32),
            pltpu.VMEM((1, N_NODES), jnp.float32),
            pltpu.VMEM((1, N_NODES), jnp.float32),
            pltpu.SemaphoreType.DMA((N_CHUNKS,)),
            pltpu.SemaphoreType.DMA((3,)),
            pltpu.SemaphoreType.DMA,
        ],
    )(t, buf, w2d)
    return (out_buf, nx2d.reshape(N_NODES))


# speedup vs baseline: 46.1510x; 46.1510x over previous
import functools

import jax
import jax.numpy as jnp
from jax.experimental import pallas as pl
from jax.experimental.pallas import tpu as pltpu

NH = 512
DT = 1.0
N_NODES = 100000
N_ROWS = NH + 2

BLOCK_R = 16


def _step_kernel(t_ref, buf_ref, bufh_ref, w_ref, out_ref, nx_ref,
                 x_s, r0_s, r1_s, row_sems):
    i = pl.program_id(0)
    tt = t_ref[0, 0]
    cx = pltpu.make_async_copy(bufh_ref.at[pl.ds(NH + tt, 1)], x_s,
                               row_sems.at[0])
    c0 = pltpu.make_async_copy(bufh_ref.at[pl.ds(tt, 1)], r0_s,
                               row_sems.at[1])
    c1 = pltpu.make_async_copy(bufh_ref.at[pl.ds(tt + 1, 1)], r1_s,
                               row_sems.at[2])

    @pl.when(i == 0)
    def _gather_and_compute():
        cx.start()
        c0.start()
        c1.start()
        cx.wait()
        c0.wait()
        c1.wait()
        x = x_s[...]
        w = w_ref[...]
        d1 = 0.1 * (r0_s[...] - x)
        xi = jnp.tanh(x + DT * d1 + w)
        d2 = 0.1 * (r1_s[...] - xi)
        nx_ref[...] = jnp.tanh(x + DT * 0.5 * (d1 + d2) + w)

    out_ref[...] = buf_ref[...]

    wrow = NH + 1 + tt

    @pl.when(i == wrow // BLOCK_R)
    def _patch():
        out_ref[pl.ds(wrow % BLOCK_R, 1), :] = nx_ref[...]


@functools.partial(jax.jit, static_argnames=())
def kernel(buf, dWt, t):
    w2d = dWt.reshape(1, N_NODES)
    grid = (pl.cdiv(N_ROWS, BLOCK_R),)
    out_buf, nx2d = pl.pallas_call(
        _step_kernel,
        grid=grid,
        in_specs=[
            pl.BlockSpec(memory_space=pltpu.SMEM),
            pl.BlockSpec((BLOCK_R, N_NODES), lambda i: (i, 0)),
            pl.BlockSpec(memory_space=pl.ANY),
            pl.BlockSpec(memory_space=pltpu.VMEM),
        ],
        out_specs=[
            pl.BlockSpec((BLOCK_R, N_NODES), lambda i: (i, 0)),
            pl.BlockSpec(memory_space=pltpu.VMEM),
        ],
        out_shape=[
            jax.ShapeDtypeStruct((N_ROWS, N_NODES), jnp.float32),
            jax.ShapeDtypeStruct((1, N_NODES), jnp.float32),
        ],
        scratch_shapes=[
            pltpu.VMEM((1, N_NODES), jnp.float32),
            pltpu.VMEM((1, N_NODES), jnp.float32),
            pltpu.VMEM((1, N_NODES), jnp.float32),
            pltpu.SemaphoreType.DMA((3,)),
        ],
    )(t, buf, buf, w2d)
    return (out_buf, nx2d.reshape(N_NODES))


# speedup vs baseline: 46.9790x; 1.0179x over previous
import functools

import jax
import jax.numpy as jnp
from jax.experimental import pallas as pl
from jax.experimental.pallas import tpu as pltpu

NH = 512
DT = 1.0
N_NODES = 100000
N_ROWS = NH + 2

BLOCK_W = 7168


def _step_kernel(t_ref, buf_ref, w_ref, out_ref, nx_ref):
    tt = t_ref[0, 0]
    out_ref[...] = buf_ref[...]
    x = buf_ref[pl.ds(NH + tt, 1), :]
    r0 = buf_ref[pl.ds(tt, 1), :]
    r1 = buf_ref[pl.ds(tt + 1, 1), :]
    w = w_ref[...]
    d1 = 0.1 * (r0 - x)
    xi = jnp.tanh(x + DT * d1 + w)
    d2 = 0.1 * (r1 - xi)
    nx = jnp.tanh(x + DT * 0.5 * (d1 + d2) + w)
    out_ref[pl.ds(NH + tt + 1, 1), :] = nx
    nx_ref[...] = nx


@functools.partial(jax.jit, static_argnames=())
def kernel(buf, dWt, t):
    w2d = dWt.reshape(1, N_NODES)
    grid = (pl.cdiv(N_NODES, BLOCK_W),)
    out_buf, nx2d = pl.pallas_call(
        _step_kernel,
        grid=grid,
        in_specs=[
            pl.BlockSpec(memory_space=pltpu.SMEM),
            pl.BlockSpec((N_ROWS, BLOCK_W), lambda j: (0, j)),
            pl.BlockSpec((1, BLOCK_W), lambda j: (0, j)),
        ],
        out_specs=[
            pl.BlockSpec((N_ROWS, BLOCK_W), lambda j: (0, j)),
            pl.BlockSpec((1, BLOCK_W), lambda j: (0, j)),
        ],
        out_shape=[
            jax.ShapeDtypeStruct((N_ROWS, N_NODES), jnp.float32),
            jax.ShapeDtypeStruct((1, N_NODES), jnp.float32),
        ],
    )(t, buf, w2d)
    return (out_buf, nx2d.reshape(N_NODES))
